# baseline (speedup 1.0000x reference)
import functools

import jax
import jax.numpy as jnp
from jax import lax
from jax.experimental import pallas as pl
from jax.experimental.pallas import tpu as pltpu
from jax.experimental.pallas import tpu_sc as plsc

B, C, N = 8, 64, 512
NUM_TYPES, D_EMB = 8, 128

_info = plsc.get_sparse_core_info()
NC, NS, L = _info.num_cores, _info.num_subcores, _info.num_lanes
NW = NC * NS

TOTAL = B * C * N * D_EMB
PER_W = TOTAL // NW
ROWS_BUF = 512
BUF_ELEMS = ROWS_BUF * D_EMB
N_WRITE = PER_W // BUF_ELEMS
N_SPMEM = 8
W_PER_B = NW // B
SLOTS = NS // W_PER_B


@functools.partial(
    pl.kernel,
    mesh=plsc.VectorSubcoreMesh(core_axis_name="c", subcore_axis_name="s"),
    compiler_params=pltpu.CompilerParams(needs_layout_passes=False),
    out_type=jax.ShapeDtypeStruct((TOTAL,), jnp.float32),
    scratch_types=[
        pltpu.VMEM((L,), jnp.int32),
        pltpu.VMEM((NUM_TYPES, D_EMB), jnp.float32),
        pltpu.VMEM((BUF_ELEMS,), jnp.float32),
        pltpu.VMEM_SHARED((SLOTS, BUF_ELEMS), jnp.float32),
        pltpu.SemaphoreType.DMA,
        pltpu.SemaphoreType.DMA,
    ],
)
def _emb_broadcast(emb_hbm, idx_hbm, out_hbm, idx_v, emb_v, rows_v, shared_v,
                   sem_w, sem_s):
    cid = lax.axis_index("c")
    sid = lax.axis_index("s")
    wid = cid * NS + sid
    my_b = wid // W_PER_B
    slot = sid // W_PER_B

    pltpu.sync_copy(idx_hbm, idx_v)
    pltpu.sync_copy(emb_hbm, emb_v)

    row_vec = plsc.load_gather(idx_v, [jnp.full((L,), my_b, jnp.int32)])
    lanes = lax.iota(jnp.int32, L)
    chunks = [
        plsc.load_gather(emb_v, [row_vec, j * L + lanes])
        for j in range(D_EMB // L)
    ]

    def fill(i, _):
        row = i * D_EMB
        for j, ch in enumerate(chunks):
            rows_v[pl.ds(row + j * L, L)] = ch
        return 0

    lax.fori_loop(0, ROWS_BUF, fill, 0)

    @pl.when(sid % W_PER_B == 0)
    def _publish():
        pltpu.sync_copy(rows_v, shared_v.at[slot])

    plsc.subcore_barrier()

    base = wid * PER_W
    writes = []
    for i in range(N_WRITE):
        dst = out_hbm.at[pl.ds(base + i * BUF_ELEMS, BUF_ELEMS)]
        if i < N_SPMEM:
            writes.append(pltpu.async_copy(shared_v.at[slot], dst, sem_s))
        else:
            writes.append(pltpu.async_copy(rows_v, dst, sem_w))
    for cp in writes:
        cp.wait()


def kernel(x, emb_table, ch_indices):
    del x
    idx16 = jnp.pad(ch_indices.astype(jnp.int32), (0, L - B))
    out = _emb_broadcast(emb_table.astype(jnp.float32), idx16)
    return out.reshape(B, C, N, D_EMB)


# speedup vs baseline: 1.0987x; 1.0987x over previous
import functools

import jax
import jax.numpy as jnp
from jax import lax
from jax.experimental import pallas as pl
from jax.experimental.pallas import tpu as pltpu
from jax.experimental.pallas import tpu_sc as plsc

B, C, N = 8, 64, 512
NUM_TYPES, D_EMB = 8, 128

_info = plsc.get_sparse_core_info()
NC, NS, L = _info.num_cores, _info.num_subcores, _info.num_lanes
NW = NC * NS

TOTAL = B * C * N * D_EMB
PER_W = TOTAL // NW
ROWS_BUF = 512
BUF_ELEMS = ROWS_BUF * D_EMB
N_WRITE = PER_W // BUF_ELEMS
HEAD_ROWS = 128
UNROLL = 4


@functools.partial(
    pl.kernel,
    mesh=plsc.VectorSubcoreMesh(core_axis_name="c", subcore_axis_name="s"),
    compiler_params=pltpu.CompilerParams(needs_layout_passes=False),
    out_type=jax.ShapeDtypeStruct((TOTAL,), jnp.float32),
    scratch_types=[
        pltpu.VMEM((L,), jnp.int32),
        pltpu.VMEM((NUM_TYPES, D_EMB), jnp.float32),
        pltpu.VMEM((BUF_ELEMS,), jnp.float32),
        pltpu.SemaphoreType.DMA,
        pltpu.SemaphoreType.DMA,
    ],
)
def _emb_broadcast(emb_hbm, idx_hbm, out_hbm, idx_v, emb_v, rows_v,
                   sem_in, sem_w):
    wid = lax.axis_index("s") * NC + lax.axis_index("c")
    my_b = wid // (NW // B)

    ins = [pltpu.async_copy(idx_hbm, idx_v, sem_in),
           pltpu.async_copy(emb_hbm, emb_v, sem_in)]
    for cp in ins:
        cp.wait()

    row_vec = plsc.load_gather(idx_v, [jnp.full((L,), my_b, jnp.int32)])
    lanes = lax.iota(jnp.int32, L)
    chunks = [
        plsc.load_gather(emb_v, [row_vec, j * L + lanes])
        for j in range(D_EMB // L)
    ]

    def fill(i, _):
        base_e = i * (UNROLL * D_EMB)
        for u in range(UNROLL):
            for j, ch in enumerate(chunks):
                rows_v[pl.ds(base_e + u * D_EMB + j * L, L)] = ch
        return 0

    base = wid * PER_W
    head_elems = HEAD_ROWS * D_EMB
    lax.fori_loop(0, HEAD_ROWS // UNROLL, fill, 0)
    writes = [
        pltpu.async_copy(rows_v.at[pl.ds(0, head_elems)],
                         out_hbm.at[pl.ds(base, head_elems)], sem_w)
    ]
    lax.fori_loop(HEAD_ROWS // UNROLL, ROWS_BUF // UNROLL, fill, 0)
    writes.append(
        pltpu.async_copy(rows_v.at[pl.ds(head_elems, BUF_ELEMS - head_elems)],
                         out_hbm.at[pl.ds(base + head_elems,
                                          BUF_ELEMS - head_elems)], sem_w))

    writes += [
        pltpu.async_copy(rows_v, out_hbm.at[pl.ds(base + i * BUF_ELEMS, BUF_ELEMS)],
                         sem_w)
        for i in range(1, N_WRITE)
    ]
    for cp in writes:
        cp.wait()


def kernel(x, emb_table, ch_indices):
    del x
    idx16 = jnp.pad(ch_indices.astype(jnp.int32), (0, L - B))
    out = _emb_broadcast(emb_table.astype(jnp.float32), idx16)
    return out.reshape(B, C, N, D_EMB)
